# baseline (speedup 1.0000x reference)
import functools
import math

import jax
import jax.numpy as jnp
from jax import lax
from jax.experimental import pallas as pl
from jax.experimental.pallas import tpu as pltpu
from jax.experimental.pallas import tpu_sc as plsc

N_NODES = 10000
D = 128

NC = 2
NS = 16
NW = NC * NS

CHUNK = 128
BLKC = 16
ROWS_PER_SUB = 640
N_PAD = NS * ROWS_PER_SUB


@functools.lru_cache(maxsize=None)
def _sc_scatter(n_chunks):
  assert n_chunks % BLKC == 0
  nb = n_chunks // BLKC
  mesh = plsc.VectorSubcoreMesh(core_axis_name="c", subcore_axis_name="s")

  @functools.partial(
      pl.kernel,
      mesh=mesh,
      out_type=jax.ShapeDtypeStruct((NC, N_PAD, D), jnp.float32),
      scratch_types=[
          pltpu.VMEM((2, BLKC, CHUNK), jnp.int32),
          pltpu.VMEM((2, BLKC, CHUNK), jnp.int32),
          pltpu.VMEM((2, CHUNK, D), jnp.float32),
          pltpu.VMEM_SHARED((N_PAD, D), jnp.float32),
          pltpu.SemaphoreType.DMA,
          pltpu.SemaphoreType.DMA,
          pltpu.SemaphoreType.DMA,
      ],
  )
  def sc_scatter(x_hbm, src_hbm, dst_hbm, out_hbm,
                 src_v, dst_v, rows_v, acc_sh, sem0, sem1, sem_idx):
    c = lax.axis_index("c")
    s = lax.axis_index("s")
    wid = s * NC + c

    zv = jnp.zeros((16,), jnp.float32)

    def zero_row(r, carry):
      for c8 in range(D // 16):
        rows_v[0, r, pl.ds(c8 * 16, 16)] = zv
      return carry

    lax.fori_loop(0, CHUNK, zero_row, 0)
    for rep in range(ROWS_PER_SUB // CHUNK):
      pltpu.sync_copy(
          rows_v.at[0],
          acc_sh.at[pl.ds(s * ROWS_PER_SUB + rep * CHUNK, CHUNK)])

    col = wid * CHUNK

    pltpu.sync_copy(src_hbm.at[pl.ds(0, BLKC), pl.ds(col, CHUNK)], src_v.at[0])
    pltpu.sync_copy(dst_hbm.at[pl.ds(0, BLKC), pl.ds(col, CHUNK)], dst_v.at[0])

    plsc.subcore_barrier()

    sems = (sem0, sem1)

    def stage_block(b):
      bb = b % 2
      h0 = pltpu.async_copy(
          src_hbm.at[pl.ds(b * BLKC, BLKC), pl.ds(col, CHUNK)],
          src_v.at[bb], sem_idx)
      h1 = pltpu.async_copy(
          dst_hbm.at[pl.ds(b * BLKC, BLKC), pl.ds(col, CHUNK)],
          dst_v.at[bb], sem_idx)
      return (h0, h1)

    def start_gather(j):
      b, k = j // BLKC, j % BLKC
      return pltpu.async_copy(x_hbm.at[src_v.at[b % 2].at[k]],
                              rows_v.at[j % 2], sems[j % 2])

    def scatter(j):
      b, k = j // BLKC, j % BLKC
      pltpu.sync_copy(rows_v.at[j % 2], acc_sh.at[dst_v.at[b % 2].at[k]],
                      add=True)

    idx_pending = stage_block(1) if nb > 1 else None
    g = [None, None]
    g[0] = start_gather(0)
    for j in range(1, n_chunks + 1):
      if j < n_chunks:
        if j % BLKC == 0:
          for h in idx_pending:
            h.wait()
        g[j % 2] = start_gather(j)
      g[(j - 1) % 2].wait()
      scatter(j - 1)
      if j < n_chunks and j % BLKC == 0:
        b = j // BLKC
        idx_pending = stage_block(b + 1) if b + 1 < nb else None

    plsc.subcore_barrier()

    pltpu.sync_copy(acc_sh.at[pl.ds(s * ROWS_PER_SUB, ROWS_PER_SUB)],
                    out_hbm.at[c].at[pl.ds(s * ROWS_PER_SUB, ROWS_PER_SUB)])

  return sc_scatter


BLK = 2000


def _tc_body(p0_ref, p1_ref, w_ref, b_ref, o_ref):
  acc = p0_ref[0] + p1_ref[0]
  o_ref[...] = (
      jnp.dot(acc, w_ref[...], preferred_element_type=jnp.float32) + b_ref[...]
  )


def _tc_finish(partials, W, b):
  grid = (N_NODES // BLK,)
  return pl.pallas_call(
      _tc_body,
      grid=grid,
      in_specs=[
          pl.BlockSpec((1, BLK, D), lambda i: (0, i, 0)),
          pl.BlockSpec((1, BLK, D), lambda i: (1, i, 0)),
          pl.BlockSpec((D, D), lambda i: (0, 0)),
          pl.BlockSpec((1, D), lambda i: (0, 0)),
      ],
      out_specs=pl.BlockSpec((BLK, D), lambda i: (i, 0)),
      out_shape=jax.ShapeDtypeStruct((N_NODES, D), jnp.float32),
  )(partials, partials, W, b.reshape(1, D))


def kernel(input, edge_index, W, b):
  E = edge_index.shape[1]
  flat = edge_index.astype(jnp.int32).reshape(-1)
  dst = flat[:E]
  src = flat[E:]
  per_blk = NW * CHUNK * BLKC
  n_chunks = BLKC * math.ceil(E / per_blk)
  e_pad = NW * n_chunks * CHUNK
  pad = e_pad - E
  if pad:
    ar = jnp.arange(pad, dtype=jnp.int32)
    src = jnp.concatenate([src, ar % N_NODES])
    dst = jnp.concatenate([dst, N_NODES + ar % (N_PAD - N_NODES)])
  src3 = src.reshape(n_chunks, NW * CHUNK)
  dst3 = dst.reshape(n_chunks, NW * CHUNK)

  partials = _sc_scatter(n_chunks)(input, src3, dst3)
  return _tc_finish(partials, W, b)


# speedup vs baseline: 1.0319x; 1.0319x over previous
import functools
import math

import jax
import jax.numpy as jnp
from jax import lax
from jax.experimental import pallas as pl
from jax.experimental.pallas import tpu as pltpu
from jax.experimental.pallas import tpu_sc as plsc

N_NODES = 10000
D = 128

NC = 2
NS = 16
NW = NC * NS

CHUNK = 128
BLKC = 16
ROWS_PER_SUB = 640
N_PAD = NS * ROWS_PER_SUB


@functools.lru_cache(maxsize=None)
def _sc_scatter(n_chunks):
  assert n_chunks % BLKC == 0
  nb = n_chunks // BLKC
  mesh = plsc.VectorSubcoreMesh(core_axis_name="c", subcore_axis_name="s")

  @functools.partial(
      pl.kernel,
      mesh=mesh,
      out_type=jax.ShapeDtypeStruct((NC, N_PAD, D), jnp.float32),
      scratch_types=[
          pltpu.VMEM((2, BLKC, CHUNK), jnp.int32),
          pltpu.VMEM((2, BLKC, CHUNK), jnp.int32),
          pltpu.VMEM((2, CHUNK, D), jnp.float32),
          pltpu.VMEM_SHARED((N_PAD, D), jnp.float32),
          pltpu.SemaphoreType.DMA,
          pltpu.SemaphoreType.DMA,
          pltpu.SemaphoreType.DMA,
      ],
  )
  def sc_scatter(x_hbm, src_hbm, dst_hbm, out_hbm,
                 src_v, dst_v, rows_v, acc_sh, sem0, sem1, sem_idx):
    c = lax.axis_index("c")
    s = lax.axis_index("s")
    wid = s * NC + c

    zv = jnp.zeros((16,), jnp.float32)

    def zero_row(r, carry):
      for c8 in range(D // 16):
        rows_v[0, r, pl.ds(c8 * 16, 16)] = zv
      return carry

    lax.fori_loop(0, CHUNK, zero_row, 0)
    for rep in range(ROWS_PER_SUB // CHUNK):
      pltpu.sync_copy(
          rows_v.at[0],
          acc_sh.at[pl.ds(s * ROWS_PER_SUB + rep * CHUNK, CHUNK)])

    col = wid * CHUNK

    pltpu.sync_copy(src_hbm.at[pl.ds(0, BLKC), pl.ds(col, CHUNK)], src_v.at[0])
    pltpu.sync_copy(dst_hbm.at[pl.ds(0, BLKC), pl.ds(col, CHUNK)], dst_v.at[0])

    plsc.subcore_barrier()

    sems = (sem0, sem1)

    def stage_block(b):
      bb = b % 2
      h0 = pltpu.async_copy(
          src_hbm.at[pl.ds(b * BLKC, BLKC), pl.ds(col, CHUNK)],
          src_v.at[bb], sem_idx)
      h1 = pltpu.async_copy(
          dst_hbm.at[pl.ds(b * BLKC, BLKC), pl.ds(col, CHUNK)],
          dst_v.at[bb], sem_idx)
      return (h0, h1)

    def start_gather(j):
      b, k = j // BLKC, j % BLKC
      return pltpu.async_copy(x_hbm.at[src_v.at[b % 2].at[k]],
                              rows_v.at[j % 2], sems[j % 2])

    def scatter(j):
      b, k = j // BLKC, j % BLKC
      pltpu.sync_copy(rows_v.at[j % 2], acc_sh.at[dst_v.at[b % 2].at[k]],
                      add=True)

    idx_pending = stage_block(1) if nb > 1 else None
    g = [None, None]
    g[0] = start_gather(0)
    for j in range(1, n_chunks + 1):
      if j < n_chunks:
        if j % BLKC == 0:
          for h in idx_pending:
            h.wait()
        g[j % 2] = start_gather(j)
      g[(j - 1) % 2].wait()
      scatter(j - 1)
      if j < n_chunks and j % BLKC == 0:
        b = j // BLKC
        idx_pending = stage_block(b + 1) if b + 1 < nb else None

    plsc.subcore_barrier()

    pltpu.sync_copy(acc_sh.at[pl.ds(s * ROWS_PER_SUB, ROWS_PER_SUB)],
                    out_hbm.at[c].at[pl.ds(s * ROWS_PER_SUB, ROWS_PER_SUB)])

  return sc_scatter


BLK = 2000


def _tc_body(p0_ref, p1_ref, w_ref, b_ref, o_ref):
  acc = p0_ref[0] + p1_ref[0]
  o_ref[...] = (
      jnp.dot(acc, w_ref[...], preferred_element_type=jnp.float32) + b_ref[...]
  )


def _tc_finish(partials, W, b):
  grid = (N_NODES // BLK,)
  return pl.pallas_call(
      _tc_body,
      grid=grid,
      in_specs=[
          pl.BlockSpec((1, BLK, D), lambda i: (0, i, 0)),
          pl.BlockSpec((1, BLK, D), lambda i: (1, i, 0)),
          pl.BlockSpec((D, D), lambda i: (0, 0)),
          pl.BlockSpec((1, D), lambda i: (0, 0)),
      ],
      out_specs=pl.BlockSpec((BLK, D), lambda i: (i, 0)),
      out_shape=jax.ShapeDtypeStruct((N_NODES, D), jnp.float32),
  )(partials, partials, W, b.reshape(1, D))


def kernel(input, edge_index, W, b):
  dst = edge_index[0].astype(jnp.int32)
  src = edge_index[1].astype(jnp.int32)
  E = src.shape[0]
  per_blk = NW * CHUNK * BLKC
  n_chunks = BLKC * math.ceil(E / per_blk)
  e_pad = NW * n_chunks * CHUNK
  pad = e_pad - E
  if pad:
    ar = jnp.arange(pad, dtype=jnp.int32)
    src = jnp.concatenate([src, ar % N_NODES])
    dst = jnp.concatenate([dst, N_NODES + ar % (N_PAD - N_NODES)])
  src3 = src.reshape(n_chunks, NW * CHUNK)
  dst3 = dst.reshape(n_chunks, NW * CHUNK)

  partials = _sc_scatter(n_chunks)(input, src3, dst3)
  return _tc_finish(partials, W, b)
